# baseline (speedup 1.0000x reference)
import jax
import jax.numpy as jnp
from jax.experimental import pallas as pl
from jax.experimental.pallas import tpu as pltpu

N = 512
D_IN = 128
D_H = 128
MSG = 32
F_HID = 32
OUT = 64
IB = 32


def _dotT(a, b):
    return jax.lax.dot_general(a, b, (((1,), (1,)), ((), ())),
                               preferred_element_type=jnp.float32)


def _rrn_kernel(x_ref, h00_ref, h01_ref, c00_ref, c01_ref, hs_ref, adj_ref,
                Wf1_ref, bf1_ref, Wf2_ref, bf2_ref,
                Wih0_ref, Whh0_ref, b0_ref,
                Wih1_ref, Whh1_ref, b1_ref,
                Wo1_ref, bo1_ref, Wo2_ref, bo2_ref,
                out_ref, hN_ref, cN_ref,
                hc_ref, gxp_ref):
    hs = hs_ref[...]
    Wf1 = Wf1_ref[...]
    Af = _dotT(hs, Wf1[:, :D_H]) + bf1_ref[...]
    Bt = jax.lax.dot_general(Wf1[:, D_H:], hs, (((1,), (1,)), ((), ())),
                             preferred_element_type=jnp.float32)

    S_t = jnp.zeros((F_HID, N), jnp.float32)
    for ib in range(N // IB):
        a_blk = Af[ib * IB:(ib + 1) * IB, :]
        adj_blk = adj_ref[ib * IB:(ib + 1) * IB, :]
        t = jnp.maximum(a_blk[:, :, None] + Bt[None, :, :], 0.0)
        t = t * adj_blk[:, None, :]
        S_t = S_t + jnp.sum(t, axis=0)

    ones_col = jnp.ones((N, 1), jnp.float32)
    csum = jax.lax.dot_general(adj_ref[...], ones_col, (((0,), (0,)), ((), ())),
                               preferred_element_type=jnp.float32)

    Wih0 = Wih0_ref[...]
    Wih0_h = Wih0[:, :D_H]
    Wih0_x = Wih0[:, D_H:2 * D_H]
    Wm = Wih0[:, 2 * D_H:]
    Wf2 = Wf2_ref[...]
    W3 = jax.lax.dot_general(Wm, Wf2, (((1,), (0,)), ((), ())),
                             preferred_element_type=jnp.float32)
    bias3 = _dotT(bf2_ref[...], Wm)

    Gx0 = (
        _dotT(hs, Wih0_h) + _dotT(x_ref[...], Wih0_x)
        + jax.lax.dot_general(S_t, W3, (((0,), (1,)), ((), ())),
                              preferred_element_type=jnp.float32)
        + csum * bias3 + b0_ref[...])

    Whh0 = Whh0_ref[...]
    Wih1 = Wih1_ref[...]
    Whh1 = Whh1_ref[...]
    b1 = b1_ref[...]

    Wa = jnp.concatenate([Whh0, Wih1], axis=0).astype(jnp.bfloat16)
    Whh1_b = Whh1.astype(jnp.bfloat16)
    Gadd = jnp.concatenate(
        [Gx0, jnp.broadcast_to(b1, (N, 4 * D_H))], axis=1)
    gxp_ref[8:8 + N - 1, :] = Gadd[1:, :]
    gxp_ref[8 + N - 1:8 + N, :] = jnp.concatenate(
        [jnp.zeros((1, 4 * D_H), jnp.float32), b1], axis=1)

    def _sig(x):
        return 0.5 * jnp.tanh(0.5 * x) + 0.5

    def lstm_step(gadd, h0c, h1c, c0c, c1c):
        d = gadd + _dotT(h0c.astype(jnp.bfloat16), Wa)
        d1 = _dotT(h1c.astype(jnp.bfloat16), Whh1_b)
        g0 = d[:, 0:4 * D_H]
        g1 = d[:, 4 * D_H:] + d1
        i0 = _sig(g0[:, 0:D_H])
        f0 = _sig(g0[:, D_H:2 * D_H])
        gg0 = jnp.tanh(g0[:, 2 * D_H:3 * D_H])
        o0 = _sig(g0[:, 3 * D_H:])
        i1 = _sig(g1[:, 0:D_H])
        f1 = _sig(g1[:, D_H:2 * D_H])
        gg1 = jnp.tanh(g1[:, 2 * D_H:3 * D_H])
        o1 = _sig(g1[:, 3 * D_H:])
        c0n = f0 * c0c + i0 * gg0
        c1n = f1 * c1c + i1 * gg1
        h0n = o0 * jnp.tanh(c0n)
        h1n = o1 * jnp.tanh(c1n)
        return h0n, h1n, c0n, c1n

    h0c, c0c = h00_ref[...], c00_ref[...]
    h1c, c1c = h01_ref[...], c01_ref[...]
    h0n, _, c0n, _ = lstm_step(Gadd[0:1, :], h0c, h1c, c0c, c1c)
    h0c, c0c = h0n, c0n

    def macro(m, carry):
        h0c, h1c, c0c, c1c = carry
        gblk = gxp_ref[pl.ds(8 * (m + 1), 8), :]
        rows = []
        for j in range(8):
            h0c, h1c, c0c, c1c = lstm_step(gblk[j:j + 1, :],
                                           h0c, h1c, c0c, c1c)
            rows.append(jnp.concatenate([h0c, h1c, c0c, c1c], axis=1))
        hc_ref[pl.ds(8 * m, 8), :] = jnp.concatenate(rows, axis=0)
        return (h0c, h1c, c0c, c1c)

    jax.lax.fori_loop(0, N // 8, macro, (h0c, h1c, c0c, c1c))

    ys1 = hc_ref[0:N, D_H:2 * D_H]
    o1 = jnp.maximum(_dotT(ys1, Wo1_ref[...]) + bo1_ref[...], 0.0)
    out_ref[...] = _dotT(o1, Wo2_ref[...]) + bo2_ref[...]
    hN_ref[0:1, :] = hc_ref[N - 2:N - 1, 0:D_H]
    hN_ref[1:2, :] = hc_ref[N - 1:N, D_H:2 * D_H]
    cN_ref[0:1, :] = hc_ref[N - 2:N - 1, 2 * D_H:3 * D_H]
    cN_ref[1:2, :] = hc_ref[N - 1:N, 3 * D_H:]


def kernel(x, h0, c0, hidden_states, adj, Wf1, bf1, Wf2, bf2,
           Wih0, Whh0, bih0, bhh0, Wih1, Whh1, bih1, bhh1,
           Wo1, bo1, Wo2, bo2):
    b0 = (bih0 + bhh0).reshape(1, -1)
    b1 = (bih1 + bhh1).reshape(1, -1)
    args = (x, h0[0:1], h0[1:2], c0[0:1], c0[1:2], hidden_states, adj,
            Wf1, bf1.reshape(1, -1), Wf2, bf2.reshape(1, -1),
            Wih0, Whh0, b0, Wih1, Whh1, b1,
            Wo1, bo1.reshape(1, -1), Wo2, bo2.reshape(1, -1))
    out, hN, cN = pl.pallas_call(
        _rrn_kernel,
        out_shape=[
            jax.ShapeDtypeStruct((N, OUT), jnp.float32),
            jax.ShapeDtypeStruct((2, D_H), jnp.float32),
            jax.ShapeDtypeStruct((2, D_H), jnp.float32),
        ],
        scratch_shapes=[
            pltpu.VMEM((N, 4 * D_H), jnp.float32),
            pltpu.VMEM((8 + N, 8 * D_H), jnp.float32),
        ],
    )(*args)
    return out, hN, cN


# speedup vs baseline: 1.0162x; 1.0162x over previous
import jax
import jax.numpy as jnp
from jax.experimental import pallas as pl
from jax.experimental.pallas import tpu as pltpu

N = 512
D_IN = 128
D_H = 128
MSG = 32
F_HID = 32
OUT = 64
IB = 32


def _dotT(a, b):
    return jax.lax.dot_general(a, b, (((1,), (1,)), ((), ())),
                               preferred_element_type=jnp.float32)


def _rrn_kernel(x_ref, h00_ref, h01_ref, c00_ref, c01_ref, hs_ref, adj_ref,
                Wf1_ref, bf1_ref, Wf2_ref, bf2_ref,
                Wih0_ref, Whh0_ref, b0_ref,
                Wih1_ref, Whh1_ref, b1_ref,
                Wo1_ref, bo1_ref, Wo2_ref, bo2_ref,
                out_ref, hN_ref, cN_ref,
                hc_ref, gxp_ref):
    hs = hs_ref[...]
    Wf1 = Wf1_ref[...]
    Af = _dotT(hs, Wf1[:, :D_H]) + bf1_ref[...]
    Bt = jax.lax.dot_general(Wf1[:, D_H:], hs, (((1,), (1,)), ((), ())),
                             preferred_element_type=jnp.float32)

    S_t = jnp.zeros((F_HID, N), jnp.float32)
    for ib in range(N // IB):
        a_blk = Af[ib * IB:(ib + 1) * IB, :]
        adj_blk = adj_ref[ib * IB:(ib + 1) * IB, :]
        t = jnp.maximum(a_blk[:, :, None] + Bt[None, :, :], 0.0)
        t = t * adj_blk[:, None, :]
        S_t = S_t + jnp.sum(t, axis=0)

    ones_col = jnp.ones((N, 1), jnp.float32)
    csum = jax.lax.dot_general(adj_ref[...], ones_col, (((0,), (0,)), ((), ())),
                               preferred_element_type=jnp.float32)

    Wih0 = Wih0_ref[...]
    Wih0_h = Wih0[:, :D_H]
    Wih0_x = Wih0[:, D_H:2 * D_H]
    Wm = Wih0[:, 2 * D_H:]
    Wf2 = Wf2_ref[...]
    W3 = jax.lax.dot_general(Wm, Wf2, (((1,), (0,)), ((), ())),
                             preferred_element_type=jnp.float32)
    bias3 = _dotT(bf2_ref[...], Wm)

    Gx0 = (
        _dotT(hs, Wih0_h) + _dotT(x_ref[...], Wih0_x)
        + jax.lax.dot_general(S_t, W3, (((0,), (1,)), ((), ())),
                              preferred_element_type=jnp.float32)
        + csum * bias3 + b0_ref[...])

    Whh0 = Whh0_ref[...]
    Wih1 = Wih1_ref[...]
    Whh1 = Whh1_ref[...]
    b1 = b1_ref[...]

    def _rw(W):
        return jnp.concatenate([0.5 * W[0:2 * D_H], 0.5 * W[3 * D_H:],
                                W[2 * D_H:3 * D_H]], axis=0)

    def _rc(A):
        return jnp.concatenate([0.5 * A[:, 0:2 * D_H], 0.5 * A[:, 3 * D_H:],
                                A[:, 2 * D_H:3 * D_H]], axis=1)

    Wa = jnp.concatenate([_rw(Whh0), _rw(Wih1)], axis=0)
    Whh1p = _rw(Whh1)
    b1p = _rc(b1)
    Gadd = jnp.concatenate(
        [_rc(Gx0), jnp.broadcast_to(b1p, (N, 4 * D_H))], axis=1)
    gxp_ref[8:8 + N - 1, :] = Gadd[1:, :]
    gxp_ref[8 + N - 1:8 + N, :] = jnp.concatenate(
        [jnp.zeros((1, 4 * D_H), jnp.float32), b1p], axis=1)

    def lstm_step(gadd, h0c, h1c, c0c, c1c):
        d = gadd + _dotT(h0c, Wa)
        d1 = _dotT(h1c, Whh1p)
        t0 = jnp.tanh(d[:, 0:4 * D_H])
        t1 = jnp.tanh(d[:, 4 * D_H:] + d1)
        i0 = 0.5 * t0[:, 0:D_H] + 0.5
        f0 = 0.5 * t0[:, D_H:2 * D_H] + 0.5
        o0 = 0.5 * t0[:, 2 * D_H:3 * D_H] + 0.5
        gg0 = t0[:, 3 * D_H:]
        i1 = 0.5 * t1[:, 0:D_H] + 0.5
        f1 = 0.5 * t1[:, D_H:2 * D_H] + 0.5
        o1 = 0.5 * t1[:, 2 * D_H:3 * D_H] + 0.5
        gg1 = t1[:, 3 * D_H:]
        c0n = f0 * c0c + i0 * gg0
        c1n = f1 * c1c + i1 * gg1
        h0n = o0 * jnp.tanh(c0n)
        h1n = o1 * jnp.tanh(c1n)
        return h0n, h1n, c0n, c1n

    h0c, c0c = h00_ref[...], c00_ref[...]
    h1c, c1c = h01_ref[...], c01_ref[...]
    h0n, _, c0n, _ = lstm_step(Gadd[0:1, :], h0c, h1c, c0c, c1c)
    h0c, c0c = h0n, c0n

    def macro(m, carry):
        h0c, h1c, c0c, c1c = carry
        gblk = gxp_ref[pl.ds(8 * (m + 1), 8), :]
        rows = []
        for j in range(8):
            h0c, h1c, c0c, c1c = lstm_step(gblk[j:j + 1, :],
                                           h0c, h1c, c0c, c1c)
            rows.append(jnp.concatenate([h0c, h1c, c0c, c1c], axis=1))
        hc_ref[pl.ds(8 * m, 8), :] = jnp.concatenate(rows, axis=0)
        return (h0c, h1c, c0c, c1c)

    jax.lax.fori_loop(0, N // 8, macro, (h0c, h1c, c0c, c1c))

    ys1 = hc_ref[0:N, D_H:2 * D_H]
    o1 = jnp.maximum(_dotT(ys1, Wo1_ref[...]) + bo1_ref[...], 0.0)
    out_ref[...] = _dotT(o1, Wo2_ref[...]) + bo2_ref[...]
    hN_ref[0:1, :] = hc_ref[N - 2:N - 1, 0:D_H]
    hN_ref[1:2, :] = hc_ref[N - 1:N, D_H:2 * D_H]
    cN_ref[0:1, :] = hc_ref[N - 2:N - 1, 2 * D_H:3 * D_H]
    cN_ref[1:2, :] = hc_ref[N - 1:N, 3 * D_H:]


def kernel(x, h0, c0, hidden_states, adj, Wf1, bf1, Wf2, bf2,
           Wih0, Whh0, bih0, bhh0, Wih1, Whh1, bih1, bhh1,
           Wo1, bo1, Wo2, bo2):
    b0 = (bih0 + bhh0).reshape(1, -1)
    b1 = (bih1 + bhh1).reshape(1, -1)
    args = (x, h0[0:1], h0[1:2], c0[0:1], c0[1:2], hidden_states, adj,
            Wf1, bf1.reshape(1, -1), Wf2, bf2.reshape(1, -1),
            Wih0, Whh0, b0, Wih1, Whh1, b1,
            Wo1, bo1.reshape(1, -1), Wo2, bo2.reshape(1, -1))
    out, hN, cN = pl.pallas_call(
        _rrn_kernel,
        out_shape=[
            jax.ShapeDtypeStruct((N, OUT), jnp.float32),
            jax.ShapeDtypeStruct((2, D_H), jnp.float32),
            jax.ShapeDtypeStruct((2, D_H), jnp.float32),
        ],
        scratch_shapes=[
            pltpu.VMEM((N, 4 * D_H), jnp.float32),
            pltpu.VMEM((8 + N, 8 * D_H), jnp.float32),
        ],
    )(*args)
    return out, hN, cN


# speedup vs baseline: 1.0540x; 1.0372x over previous
import jax
import jax.numpy as jnp
from jax.experimental import pallas as pl
from jax.experimental.pallas import tpu as pltpu

N = 512
D_IN = 128
D_H = 128
MSG = 32
F_HID = 32
OUT = 64
IB = 32
U = 16


def _dotT(a, b):
    return jax.lax.dot_general(a, b, (((1,), (1,)), ((), ())),
                               preferred_element_type=jnp.float32)


def _rrn_kernel(x_ref, h00_ref, h01_ref, c00_ref, c01_ref, hs_ref, adj_ref,
                Wf1_ref, bf1_ref, Wf2_ref, bf2_ref,
                Wih0_ref, Whh0_ref, b0_ref,
                Wih1_ref, Whh1_ref, b1_ref,
                Wo1_ref, bo1_ref, Wo2_ref, bo2_ref,
                out_ref, hN_ref, cN_ref,
                hc_ref, gxp_ref):
    hs = hs_ref[...]
    Wf1 = Wf1_ref[...]
    Af = _dotT(hs, Wf1[:, :D_H]) + bf1_ref[...]
    Bt = jax.lax.dot_general(Wf1[:, D_H:], hs, (((1,), (1,)), ((), ())),
                             preferred_element_type=jnp.float32)

    S_t = jnp.zeros((F_HID, N), jnp.float32)
    for ib in range(N // IB):
        a_blk = Af[ib * IB:(ib + 1) * IB, :]
        adj_blk = adj_ref[ib * IB:(ib + 1) * IB, :]
        t = jnp.maximum(a_blk[:, :, None] + Bt[None, :, :], 0.0)
        t = t * adj_blk[:, None, :]
        S_t = S_t + jnp.sum(t, axis=0)

    ones_col = jnp.ones((N, 1), jnp.float32)
    csum = jax.lax.dot_general(adj_ref[...], ones_col, (((0,), (0,)), ((), ())),
                               preferred_element_type=jnp.float32)

    Wih0 = Wih0_ref[...]
    Wih0_h = Wih0[:, :D_H]
    Wih0_x = Wih0[:, D_H:2 * D_H]
    Wm = Wih0[:, 2 * D_H:]
    Wf2 = Wf2_ref[...]
    W3 = jax.lax.dot_general(Wm, Wf2, (((1,), (0,)), ((), ())),
                             preferred_element_type=jnp.float32)
    bias3 = _dotT(bf2_ref[...], Wm)

    Gx0 = (
        _dotT(hs, Wih0_h) + _dotT(x_ref[...], Wih0_x)
        + jax.lax.dot_general(S_t, W3, (((0,), (1,)), ((), ())),
                              preferred_element_type=jnp.float32)
        + csum * bias3 + b0_ref[...])

    Whh0 = Whh0_ref[...]
    Wih1 = Wih1_ref[...]
    Whh1 = Whh1_ref[...]
    b1 = b1_ref[...]

    def _rw(W):
        return jnp.concatenate([0.5 * W[0:2 * D_H], 0.5 * W[3 * D_H:],
                                W[2 * D_H:3 * D_H]], axis=0)

    def _rc(A):
        return jnp.concatenate([0.5 * A[:, 0:2 * D_H], 0.5 * A[:, 3 * D_H:],
                                A[:, 2 * D_H:3 * D_H]], axis=1)

    Wa = jnp.concatenate([_rw(Whh0), _rw(Wih1)], axis=0)
    Whh1p = _rw(Whh1)
    b1p = _rc(b1)
    Gadd = jnp.concatenate(
        [_rc(Gx0), jnp.broadcast_to(b1p, (N, 4 * D_H))], axis=1)
    gxp_ref[U:U + N - 1, :] = Gadd[1:, :]
    gxp_ref[U + N - 1:U + N, :] = jnp.concatenate(
        [jnp.zeros((1, 4 * D_H), jnp.float32), b1p], axis=1)

    def lstm_step(gadd, h0c, h1c, c0c, c1c):
        d = gadd + _dotT(h0c, Wa)
        d1 = _dotT(h1c, Whh1p)
        t0 = jnp.tanh(d[:, 0:4 * D_H])
        t1 = jnp.tanh(d[:, 4 * D_H:] + d1)
        i0 = 0.5 * t0[:, 0:D_H] + 0.5
        f0 = 0.5 * t0[:, D_H:2 * D_H] + 0.5
        o0 = 0.5 * t0[:, 2 * D_H:3 * D_H] + 0.5
        gg0 = t0[:, 3 * D_H:]
        i1 = 0.5 * t1[:, 0:D_H] + 0.5
        f1 = 0.5 * t1[:, D_H:2 * D_H] + 0.5
        o1 = 0.5 * t1[:, 2 * D_H:3 * D_H] + 0.5
        gg1 = t1[:, 3 * D_H:]
        c0n = f0 * c0c + i0 * gg0
        c1n = f1 * c1c + i1 * gg1
        h0n = o0 * jnp.tanh(c0n)
        h1n = o1 * jnp.tanh(c1n)
        return h0n, h1n, c0n, c1n

    h0c, c0c = h00_ref[...], c00_ref[...]
    h1c, c1c = h01_ref[...], c01_ref[...]
    h0n, _, c0n, _ = lstm_step(Gadd[0:1, :], h0c, h1c, c0c, c1c)
    h0c, c0c = h0n, c0n

    def macro(m, carry):
        h0c, h1c, c0c, c1c = carry
        gblk = gxp_ref[pl.ds(U * (m + 1), U), :]
        rows = []
        for j in range(U):
            h0c, h1c, c0c, c1c = lstm_step(gblk[j:j + 1, :],
                                           h0c, h1c, c0c, c1c)
            rows.append(jnp.concatenate([h0c, h1c, c0c, c1c], axis=1))
        hc_ref[pl.ds(U * m, U), :] = jnp.concatenate(rows, axis=0)
        return (h0c, h1c, c0c, c1c)

    jax.lax.fori_loop(0, N // U, macro, (h0c, h1c, c0c, c1c))

    ys1 = hc_ref[0:N, D_H:2 * D_H]
    o1 = jnp.maximum(_dotT(ys1, Wo1_ref[...]) + bo1_ref[...], 0.0)
    out_ref[...] = _dotT(o1, Wo2_ref[...]) + bo2_ref[...]
    hN_ref[0:1, :] = hc_ref[N - 2:N - 1, 0:D_H]
    hN_ref[1:2, :] = hc_ref[N - 1:N, D_H:2 * D_H]
    cN_ref[0:1, :] = hc_ref[N - 2:N - 1, 2 * D_H:3 * D_H]
    cN_ref[1:2, :] = hc_ref[N - 1:N, 3 * D_H:]


def kernel(x, h0, c0, hidden_states, adj, Wf1, bf1, Wf2, bf2,
           Wih0, Whh0, bih0, bhh0, Wih1, Whh1, bih1, bhh1,
           Wo1, bo1, Wo2, bo2):
    b0 = (bih0 + bhh0).reshape(1, -1)
    b1 = (bih1 + bhh1).reshape(1, -1)
    args = (x, h0[0:1], h0[1:2], c0[0:1], c0[1:2], hidden_states, adj,
            Wf1, bf1.reshape(1, -1), Wf2, bf2.reshape(1, -1),
            Wih0, Whh0, b0, Wih1, Whh1, b1,
            Wo1, bo1.reshape(1, -1), Wo2, bo2.reshape(1, -1))
    out, hN, cN = pl.pallas_call(
        _rrn_kernel,
        out_shape=[
            jax.ShapeDtypeStruct((N, OUT), jnp.float32),
            jax.ShapeDtypeStruct((2, D_H), jnp.float32),
            jax.ShapeDtypeStruct((2, D_H), jnp.float32),
        ],
        scratch_shapes=[
            pltpu.VMEM((N, 4 * D_H), jnp.float32),
            pltpu.VMEM((U + N, 8 * D_H), jnp.float32),
        ],
    )(*args)
    return out, hN, cN


# speedup vs baseline: 1.0735x; 1.0185x over previous
import jax
import jax.numpy as jnp
from jax.experimental import pallas as pl
from jax.experimental.pallas import tpu as pltpu

N = 512
D_IN = 128
D_H = 128
MSG = 32
F_HID = 32
OUT = 64
IB = 32
U = 32


def _dotT(a, b):
    return jax.lax.dot_general(a, b, (((1,), (1,)), ((), ())),
                               preferred_element_type=jnp.float32)


def _rrn_kernel(x_ref, h00_ref, h01_ref, c00_ref, c01_ref, hs_ref, adj_ref,
                Wf1_ref, bf1_ref, Wf2_ref, bf2_ref,
                Wih0_ref, Whh0_ref, b0_ref,
                Wih1_ref, Whh1_ref, b1_ref,
                Wo1_ref, bo1_ref, Wo2_ref, bo2_ref,
                out_ref, hN_ref, cN_ref,
                hc_ref, gxp_ref):
    hs = hs_ref[...]
    Wf1 = Wf1_ref[...]
    Af = _dotT(hs, Wf1[:, :D_H]) + bf1_ref[...]
    Bt = jax.lax.dot_general(Wf1[:, D_H:], hs, (((1,), (1,)), ((), ())),
                             preferred_element_type=jnp.float32)

    S_t = jnp.zeros((F_HID, N), jnp.float32)
    for ib in range(N // IB):
        a_blk = Af[ib * IB:(ib + 1) * IB, :]
        adj_blk = adj_ref[ib * IB:(ib + 1) * IB, :]
        t = jnp.maximum(a_blk[:, :, None] + Bt[None, :, :], 0.0)
        t = t * adj_blk[:, None, :]
        S_t = S_t + jnp.sum(t, axis=0)

    ones_col = jnp.ones((N, 1), jnp.float32)
    csum = jax.lax.dot_general(adj_ref[...], ones_col, (((0,), (0,)), ((), ())),
                               preferred_element_type=jnp.float32)

    Wih0 = Wih0_ref[...]
    Wih0_h = Wih0[:, :D_H]
    Wih0_x = Wih0[:, D_H:2 * D_H]
    Wm = Wih0[:, 2 * D_H:]
    Wf2 = Wf2_ref[...]
    W3 = jax.lax.dot_general(Wm, Wf2, (((1,), (0,)), ((), ())),
                             preferred_element_type=jnp.float32)
    bias3 = _dotT(bf2_ref[...], Wm)

    Gx0 = (
        _dotT(hs, Wih0_h) + _dotT(x_ref[...], Wih0_x)
        + jax.lax.dot_general(S_t, W3, (((0,), (1,)), ((), ())),
                              preferred_element_type=jnp.float32)
        + csum * bias3 + b0_ref[...])

    Whh0 = Whh0_ref[...]
    Wih1 = Wih1_ref[...]
    Whh1 = Whh1_ref[...]
    b1 = b1_ref[...]

    def _rw(W):
        return jnp.concatenate([0.5 * W[0:2 * D_H], 0.5 * W[3 * D_H:],
                                W[2 * D_H:3 * D_H]], axis=0)

    def _rc(A):
        return jnp.concatenate([0.5 * A[:, 0:2 * D_H], 0.5 * A[:, 3 * D_H:],
                                A[:, 2 * D_H:3 * D_H]], axis=1)

    Wa = jnp.concatenate([_rw(Whh0), _rw(Wih1)], axis=0)
    Whh1p = _rw(Whh1)
    b1p = _rc(b1)
    Gadd = jnp.concatenate(
        [_rc(Gx0), jnp.broadcast_to(b1p, (N, 4 * D_H))], axis=1)
    gxp_ref[U:U + N - 1, :] = Gadd[1:, :]
    gxp_ref[U + N - 1:U + N, :] = jnp.concatenate(
        [jnp.zeros((1, 4 * D_H), jnp.float32), b1p], axis=1)

    def lstm_step(gadd, h0c, h1c, c0c, c1c):
        d = gadd + _dotT(h0c, Wa)
        d1 = _dotT(h1c, Whh1p)
        t0 = jnp.tanh(d[:, 0:4 * D_H])
        t1 = jnp.tanh(d[:, 4 * D_H:] + d1)
        i0 = 0.5 * t0[:, 0:D_H] + 0.5
        f0 = 0.5 * t0[:, D_H:2 * D_H] + 0.5
        o0 = 0.5 * t0[:, 2 * D_H:3 * D_H] + 0.5
        gg0 = t0[:, 3 * D_H:]
        i1 = 0.5 * t1[:, 0:D_H] + 0.5
        f1 = 0.5 * t1[:, D_H:2 * D_H] + 0.5
        o1 = 0.5 * t1[:, 2 * D_H:3 * D_H] + 0.5
        gg1 = t1[:, 3 * D_H:]
        c0n = f0 * c0c + i0 * gg0
        c1n = f1 * c1c + i1 * gg1
        h0n = o0 * jnp.tanh(c0n)
        h1n = o1 * jnp.tanh(c1n)
        return h0n, h1n, c0n, c1n

    h0c, c0c = h00_ref[...], c00_ref[...]
    h1c, c1c = h01_ref[...], c01_ref[...]
    h0n, _, c0n, _ = lstm_step(Gadd[0:1, :], h0c, h1c, c0c, c1c)
    h0c, c0c = h0n, c0n

    def macro(m, carry):
        h0c, h1c, c0c, c1c = carry
        gblk = gxp_ref[pl.ds(U * (m + 1), U), :]
        rows = []
        for j in range(U):
            h0c, h1c, c0c, c1c = lstm_step(gblk[j:j + 1, :],
                                           h0c, h1c, c0c, c1c)
            rows.append(jnp.concatenate([h0c, h1c, c0c, c1c], axis=1))
        hc_ref[pl.ds(U * m, U), :] = jnp.concatenate(rows, axis=0)
        return (h0c, h1c, c0c, c1c)

    jax.lax.fori_loop(0, N // U, macro, (h0c, h1c, c0c, c1c))

    ys1 = hc_ref[0:N, D_H:2 * D_H]
    o1 = jnp.maximum(_dotT(ys1, Wo1_ref[...]) + bo1_ref[...], 0.0)
    out_ref[...] = _dotT(o1, Wo2_ref[...]) + bo2_ref[...]
    hN_ref[0:1, :] = hc_ref[N - 2:N - 1, 0:D_H]
    hN_ref[1:2, :] = hc_ref[N - 1:N, D_H:2 * D_H]
    cN_ref[0:1, :] = hc_ref[N - 2:N - 1, 2 * D_H:3 * D_H]
    cN_ref[1:2, :] = hc_ref[N - 1:N, 3 * D_H:]


def kernel(x, h0, c0, hidden_states, adj, Wf1, bf1, Wf2, bf2,
           Wih0, Whh0, bih0, bhh0, Wih1, Whh1, bih1, bhh1,
           Wo1, bo1, Wo2, bo2):
    b0 = (bih0 + bhh0).reshape(1, -1)
    b1 = (bih1 + bhh1).reshape(1, -1)
    args = (x, h0[0:1], h0[1:2], c0[0:1], c0[1:2], hidden_states, adj,
            Wf1, bf1.reshape(1, -1), Wf2, bf2.reshape(1, -1),
            Wih0, Whh0, b0, Wih1, Whh1, b1,
            Wo1, bo1.reshape(1, -1), Wo2, bo2.reshape(1, -1))
    out, hN, cN = pl.pallas_call(
        _rrn_kernel,
        out_shape=[
            jax.ShapeDtypeStruct((N, OUT), jnp.float32),
            jax.ShapeDtypeStruct((2, D_H), jnp.float32),
            jax.ShapeDtypeStruct((2, D_H), jnp.float32),
        ],
        scratch_shapes=[
            pltpu.VMEM((N, 4 * D_H), jnp.float32),
            pltpu.VMEM((U + N, 8 * D_H), jnp.float32),
        ],
    )(*args)
    return out, hN, cN


# speedup vs baseline: 1.0866x; 1.0122x over previous
import jax
import jax.numpy as jnp
from jax.experimental import pallas as pl
from jax.experimental.pallas import tpu as pltpu

N = 512
D_IN = 128
D_H = 128
MSG = 32
F_HID = 32
OUT = 64
IB = 32
U = 64


def _dotT(a, b):
    return jax.lax.dot_general(a, b, (((1,), (1,)), ((), ())),
                               preferred_element_type=jnp.float32)


def _rrn_kernel(x_ref, h00_ref, h01_ref, c00_ref, c01_ref, hs_ref, adj_ref,
                Wf1_ref, bf1_ref, Wf2_ref, bf2_ref,
                Wih0_ref, Whh0_ref, b0_ref,
                Wih1_ref, Whh1_ref, b1_ref,
                Wo1_ref, bo1_ref, Wo2_ref, bo2_ref,
                out_ref, hN_ref, cN_ref,
                hc_ref, gxp_ref):
    hs = hs_ref[...]
    Wf1 = Wf1_ref[...]
    Af = _dotT(hs, Wf1[:, :D_H]) + bf1_ref[...]
    Bt = jax.lax.dot_general(Wf1[:, D_H:], hs, (((1,), (1,)), ((), ())),
                             preferred_element_type=jnp.float32)

    S_t = jnp.zeros((F_HID, N), jnp.float32)
    for ib in range(N // IB):
        a_blk = Af[ib * IB:(ib + 1) * IB, :]
        adj_blk = adj_ref[ib * IB:(ib + 1) * IB, :]
        t = jnp.maximum(a_blk[:, :, None] + Bt[None, :, :], 0.0)
        t = t * adj_blk[:, None, :]
        S_t = S_t + jnp.sum(t, axis=0)

    ones_col = jnp.ones((N, 1), jnp.float32)
    csum = jax.lax.dot_general(adj_ref[...], ones_col, (((0,), (0,)), ((), ())),
                               preferred_element_type=jnp.float32)

    Wih0 = Wih0_ref[...]
    Wih0_h = Wih0[:, :D_H]
    Wih0_x = Wih0[:, D_H:2 * D_H]
    Wm = Wih0[:, 2 * D_H:]
    Wf2 = Wf2_ref[...]
    W3 = jax.lax.dot_general(Wm, Wf2, (((1,), (0,)), ((), ())),
                             preferred_element_type=jnp.float32)
    bias3 = _dotT(bf2_ref[...], Wm)

    Gx0 = (
        _dotT(hs, Wih0_h) + _dotT(x_ref[...], Wih0_x)
        + jax.lax.dot_general(S_t, W3, (((0,), (1,)), ((), ())),
                              preferred_element_type=jnp.float32)
        + csum * bias3 + b0_ref[...])

    Whh0 = Whh0_ref[...]
    Wih1 = Wih1_ref[...]
    Whh1 = Whh1_ref[...]
    b1 = b1_ref[...]

    def _rw(W):
        return jnp.concatenate([0.5 * W[0:2 * D_H], 0.5 * W[3 * D_H:],
                                W[2 * D_H:3 * D_H]], axis=0)

    def _rc(A):
        return jnp.concatenate([0.5 * A[:, 0:2 * D_H], 0.5 * A[:, 3 * D_H:],
                                A[:, 2 * D_H:3 * D_H]], axis=1)

    Wa = jnp.concatenate([_rw(Whh0), _rw(Wih1)], axis=0)
    Whh1p = _rw(Whh1)
    b1p = _rc(b1)
    Gadd = jnp.concatenate(
        [_rc(Gx0), jnp.broadcast_to(b1p, (N, 4 * D_H))], axis=1)
    gxp_ref[U:U + N - 1, :] = Gadd[1:, :]
    gxp_ref[U + N - 1:U + N, :] = jnp.concatenate(
        [jnp.zeros((1, 4 * D_H), jnp.float32), b1p], axis=1)

    def lstm_step(gadd, h0c, h1c, c0c, c1c):
        d = gadd + _dotT(h0c, Wa)
        d1 = _dotT(h1c, Whh1p)
        t0 = jnp.tanh(d[:, 0:4 * D_H])
        t1 = jnp.tanh(d[:, 4 * D_H:] + d1)
        i0 = 0.5 * t0[:, 0:D_H] + 0.5
        f0 = 0.5 * t0[:, D_H:2 * D_H] + 0.5
        o0 = 0.5 * t0[:, 2 * D_H:3 * D_H] + 0.5
        gg0 = t0[:, 3 * D_H:]
        i1 = 0.5 * t1[:, 0:D_H] + 0.5
        f1 = 0.5 * t1[:, D_H:2 * D_H] + 0.5
        o1 = 0.5 * t1[:, 2 * D_H:3 * D_H] + 0.5
        gg1 = t1[:, 3 * D_H:]
        c0n = f0 * c0c + i0 * gg0
        c1n = f1 * c1c + i1 * gg1
        h0n = o0 * jnp.tanh(c0n)
        h1n = o1 * jnp.tanh(c1n)
        return h0n, h1n, c0n, c1n

    h0c, c0c = h00_ref[...], c00_ref[...]
    h1c, c1c = h01_ref[...], c01_ref[...]
    h0n, _, c0n, _ = lstm_step(Gadd[0:1, :], h0c, h1c, c0c, c1c)
    h0c, c0c = h0n, c0n

    def macro(m, carry):
        h0c, h1c, c0c, c1c = carry
        gblk = gxp_ref[pl.ds(U * (m + 1), U), :]
        rows = []
        for j in range(U):
            h0c, h1c, c0c, c1c = lstm_step(gblk[j:j + 1, :],
                                           h0c, h1c, c0c, c1c)
            rows.append(jnp.concatenate([h0c, h1c, c0c, c1c], axis=1))
        hc_ref[pl.ds(U * m, U), :] = jnp.concatenate(rows, axis=0)
        return (h0c, h1c, c0c, c1c)

    jax.lax.fori_loop(0, N // U, macro, (h0c, h1c, c0c, c1c))

    ys1 = hc_ref[0:N, D_H:2 * D_H]
    o1 = jnp.maximum(_dotT(ys1, Wo1_ref[...]) + bo1_ref[...], 0.0)
    out_ref[...] = _dotT(o1, Wo2_ref[...]) + bo2_ref[...]
    hN_ref[0:1, :] = hc_ref[N - 2:N - 1, 0:D_H]
    hN_ref[1:2, :] = hc_ref[N - 1:N, D_H:2 * D_H]
    cN_ref[0:1, :] = hc_ref[N - 2:N - 1, 2 * D_H:3 * D_H]
    cN_ref[1:2, :] = hc_ref[N - 1:N, 3 * D_H:]


def kernel(x, h0, c0, hidden_states, adj, Wf1, bf1, Wf2, bf2,
           Wih0, Whh0, bih0, bhh0, Wih1, Whh1, bih1, bhh1,
           Wo1, bo1, Wo2, bo2):
    b0 = (bih0 + bhh0).reshape(1, -1)
    b1 = (bih1 + bhh1).reshape(1, -1)
    args = (x, h0[0:1], h0[1:2], c0[0:1], c0[1:2], hidden_states, adj,
            Wf1, bf1.reshape(1, -1), Wf2, bf2.reshape(1, -1),
            Wih0, Whh0, b0, Wih1, Whh1, b1,
            Wo1, bo1.reshape(1, -1), Wo2, bo2.reshape(1, -1))
    out, hN, cN = pl.pallas_call(
        _rrn_kernel,
        out_shape=[
            jax.ShapeDtypeStruct((N, OUT), jnp.float32),
            jax.ShapeDtypeStruct((2, D_H), jnp.float32),
            jax.ShapeDtypeStruct((2, D_H), jnp.float32),
        ],
        scratch_shapes=[
            pltpu.VMEM((N, 4 * D_H), jnp.float32),
            pltpu.VMEM((U + N, 8 * D_H), jnp.float32),
        ],
    )(*args)
    return out, hN, cN
